# baseline (speedup 1.0000x reference)
import functools

import jax
import jax.numpy as jnp
import numpy as np
from jax import lax
from jax.experimental import pallas as pl
from jax.experimental.pallas import tpu as pltpu
from jax.experimental.pallas import tpu_sc as plsc

N = 10000
E = 160000
IN = 128
H = 16
OUT = 2

NW = 32
CH = 128
EP = 163840
NCH = EP // (NW * CH)
PER = EP // NW
NACC = 10240
RPT = NACC // 16
NRC = RPT // CH
EC = 64
ENC = PER // EC
BEA = 1024
NHI = NACC // CH
AW = 24

def _mesh():
    return plsc.VectorSubcoreMesh(core_axis_name="c", subcore_axis_name="s")


def _make_gather(D):
    @functools.partial(
        pl.kernel,
        mesh=_mesh(),
        out_type=jax.ShapeDtypeStruct((EP, D), jnp.float32),
        scratch_types=[
            pltpu.VMEM((EC,), jnp.int32),
            pltpu.VMEM((EC, D), jnp.float32),
            pltpu.SemaphoreType.DMA,
        ],
    )
    def gather(table_hbm, idx_hbm, out_hbm, idx_c, buf_v, sem):
        wid = lax.axis_index("s") * 2 + lax.axis_index("c")

        def body(j, carry):
            pltpu.sync_copy(idx_hbm.at[wid, j], idx_c)
            pltpu.async_copy(table_hbm.at[idx_c], buf_v, sem).wait()
            pltpu.sync_copy(buf_v, out_hbm.at[pl.ds(wid * PER + j * EC, EC)])
            return carry

        lax.fori_loop(0, ENC, body, 0)

    return gather


def _msg1_body(xj_ref, ea_ref, w_ref, b_ref, out_ref):
    ea = ea_ref[...]
    wf = jnp.maximum(
        jnp.dot(ea, w_ref[...], preferred_element_type=jnp.float32)
        + b_ref[...], 0.0)
    xj = xj_ref[...]
    cols = []
    for o in range(H):
        p = wf[:, o * IN:(o + 1) * IN] * xj
        cols.append(jnp.sum(p, axis=1, keepdims=True))
    out_ref[...] = jnp.concatenate(cols, axis=1)


def _msg1(xj, ea, wr, br):
    BE = 512
    return pl.pallas_call(
        _msg1_body,
        grid=(EP // BE,),
        in_specs=[
            pl.BlockSpec((BE, IN), lambda i: (i, 0)),
            pl.BlockSpec((BE, 2), lambda i: (i, 0)),
            pl.BlockSpec((2, IN * H), lambda i: (0, 0)),
            pl.BlockSpec((1, IN * H), lambda i: (0, 0)),
        ],
        out_specs=pl.BlockSpec((BE, H), lambda i: (i, 0)),
        out_shape=jax.ShapeDtypeStruct((EP, H), jnp.float32),
    )(xj, ea, wr, br)


def _make_msg_small(oc, ow):
    def body(xj_ref, ea_ref, w_ref, b_ref, rep_ref, sel_ref, out_ref):
        wf = jnp.maximum(
            jnp.dot(ea_ref[...], w_ref[...],
                    preferred_element_type=jnp.float32) + b_ref[...], 0.0)
        xr = jnp.dot(xj_ref[:, :H], rep_ref[...],
                     preferred_element_type=jnp.float32)
        out_ref[...] = jnp.dot(wf * xr, sel_ref[...],
                               preferred_element_type=jnp.float32)

    BE = 2048
    K = H * oc

    def run(xj, ea, w, b, rep, sel):
        return pl.pallas_call(
            body,
            grid=(EP // BE,),
            in_specs=[
                pl.BlockSpec((BE, IN), lambda i: (i, 0)),
                pl.BlockSpec((BE, 2), lambda i: (i, 0)),
                pl.BlockSpec((2, K), lambda i: (0, 0)),
                pl.BlockSpec((1, K), lambda i: (0, 0)),
                pl.BlockSpec((H, K), lambda i: (0, 0)),
                pl.BlockSpec((K, ow), lambda i: (0, 0)),
            ],
            out_specs=pl.BlockSpec((BE, H), lambda i: (i, 0)),
            out_shape=jax.ShapeDtypeStruct((EP, H), jnp.float32),
        )(xj, ea, w, b, rep, sel)

    return run


_msg2 = _make_msg_small(H, H)
_msg3 = _make_msg_small(OUT, H)


def _agg_body(msg_ref, dstr_ref, dstc_ref, out_ref):
    i = pl.program_id(0)

    @pl.when(i == 0)
    def _():
        out_ref[...] = jnp.zeros((NACC, AW), jnp.float32)

    msg = msg_ref[...]
    m24 = jnp.concatenate(
        [msg, jnp.ones((BEA, 1), jnp.float32),
         jnp.zeros((BEA, AW - H - 1), jnp.float32)], axis=1)
    dr = dstr_ref[0]
    dc = dstc_ref[0]
    lo = jax.lax.rem(dr, CH)
    ohT = (jax.lax.broadcasted_iota(jnp.int32, (CH, BEA), 0) ==
           lo).astype(jnp.float32)
    hic = dc // CH
    for hi in range(NHI):
        mh = m24 * (hic == hi).astype(jnp.float32)
        blk = jnp.dot(ohT, mh, preferred_element_type=jnp.float32)
        out_ref[pl.ds(hi * CH, CH), :] += blk


def _agg(msg, dstr, dstc):
    return pl.pallas_call(
        _agg_body,
        grid=(EP // BEA,),
        in_specs=[
            pl.BlockSpec((BEA, H), lambda i: (i, 0)),
            pl.BlockSpec((1, 1, BEA), lambda i: (i, 0, 0)),
            pl.BlockSpec((1, BEA, 1), lambda i: (i, 0, 0)),
        ],
        out_specs=pl.BlockSpec((NACC, AW), lambda i: (0, 0)),
        out_shape=jax.ShapeDtypeStruct((NACC, AW), jnp.float32),
    )(msg, dstr, dstc)


def _make_finalize(ric, oc, do_relu, ow):
    R = 1024

    def body(acc_ref, hp_ref, root_ref, bias_ref, out_ref):
        acc = acc_ref[...]
        cnt = acc[:, H:H + 1]
        agg = acc[:, :oc] / jnp.maximum(cnt, 1.0)
        h = agg + jnp.dot(hp_ref[:, :ric], root_ref[...],
                          preferred_element_type=jnp.float32) + bias_ref[...]
        h = jnp.maximum(h, 0.0) if do_relu else h
        if ow > oc:
            h = jnp.concatenate(
                [h, jnp.zeros((h.shape[0], ow - oc), jnp.float32)], axis=1)
        out_ref[...] = h

    def run(acc, hp, root, bias):
        return pl.pallas_call(
            body,
            grid=(NACC // R,),
            in_specs=[
                pl.BlockSpec((R, AW), lambda i: (i, 0)),
                pl.BlockSpec((R, IN), lambda i: (i, 0)),
                pl.BlockSpec((ric, oc), lambda i: (0, 0)),
                pl.BlockSpec((1, oc), lambda i: (0, 0)),
            ],
            out_specs=pl.BlockSpec((R, ow), lambda i: (i, 0)),
            out_shape=jax.ShapeDtypeStruct((NACC, ow), jnp.float32),
        )(acc, hp, root, bias)

    return run


_fin1 = _make_finalize(IN, H, True, IN)
_fin2 = _make_finalize(H, H, True, IN)
_fin3 = _make_finalize(H, OUT, False, OUT)

_REP2 = np.kron(np.eye(H), np.ones((1, H))).astype(np.float32)
_SEL2 = np.kron(np.ones((H, 1)), np.eye(H)).astype(np.float32)
_REP3 = np.kron(np.eye(H), np.ones((1, OUT))).astype(np.float32)
_SEL3 = np.zeros((H * OUT, H), np.float32)
_SEL3[:, :OUT] = np.kron(np.ones((H, 1)), np.eye(OUT))


def kernel(x, edge_index, edge_attr, W1, b1, W2, b2, W3, b3,
           root1, bias1, root2, bias2, root3, bias3):
    src = jnp.pad(edge_index[0], (0, EP - E)).reshape(NW, ENC, EC)
    dst_flat = jnp.pad(edge_index[1], (0, EP - E), constant_values=N)
    dstr = dst_flat.reshape(EP // BEA, 1, BEA)
    dstc = dst_flat.reshape(EP // BEA, BEA, 1)
    ea = jnp.pad(edge_attr, ((0, EP - E), (0, 0)))
    x_pad = jnp.pad(x, ((0, NACC - N), (0, 0)))

    wr1 = W1.reshape(2, IN, H).transpose(0, 2, 1).reshape(2, IN * H)
    br1 = b1.reshape(IN, H).T.reshape(1, IN * H)

    _gather128 = _make_gather(IN)

    xj = _gather128(x, src)
    m1 = _msg1(xj, ea, wr1, br1)
    acc1 = _agg(m1, dstr, dstc)
    h1 = _fin1(acc1, x_pad, root1, bias1.reshape(1, H))

    xj2 = _gather128(h1, src)
    m2 = _msg2(xj2, ea, W2, b2.reshape(1, H * H), _REP2, _SEL2)
    acc2 = _agg(m2, dstr, dstc)
    h2 = _fin2(acc2, h1, root2, bias2.reshape(1, H))

    xj3 = _gather128(h2, src)
    m3 = _msg3(xj3, ea, W3, b3.reshape(1, H * OUT), _REP3, _SEL3)
    acc3 = _agg(m3, dstr, dstc)
    out = _fin3(acc3, h2, root3, bias3.reshape(1, OUT))
    return out[:N]


# speedup vs baseline: 1.2109x; 1.2109x over previous
import functools

import jax
import jax.numpy as jnp
import numpy as np
from jax import lax
from jax.experimental import pallas as pl
from jax.experimental.pallas import tpu as pltpu
from jax.experimental.pallas import tpu_sc as plsc

N = 10000
E = 160000
IN = 128
H = 16
OUT = 2

NW = 32
CH = 128
EP = 163840
NCH = EP // (NW * CH)
PER = EP // NW
NACC = 10240
RPT = NACC // 16
NRC = RPT // CH
EC = 128
ENC = PER // EC
BEA = 512
LO = 256
NHI = NACC // LO
AW = 32

def _mesh():
    return plsc.VectorSubcoreMesh(core_axis_name="c", subcore_axis_name="s")


def _make_gather(D):
    @functools.partial(
        pl.kernel,
        mesh=_mesh(),
        out_type=jax.ShapeDtypeStruct((EP, D), jnp.float32),
        scratch_types=[
            pltpu.VMEM((ENC, EC), jnp.int32),
            pltpu.VMEM((EC, D), jnp.float32),
            pltpu.VMEM((EC, D), jnp.float32),
            pltpu.SemaphoreType.DMA,
            pltpu.SemaphoreType.DMA,
        ],
    )
    def gather(table_hbm, idx_hbm, out_hbm, idx_v, buf0, buf1, sem0, sem1):
        wid = lax.axis_index("s") * 2 + lax.axis_index("c")
        pltpu.sync_copy(idx_hbm.at[wid], idx_v)
        bufs = (buf0, buf1)
        sems = (sem0, sem1)
        pltpu.async_copy(table_hbm.at[idx_v.at[0]], buf0, sem0)

        def body(jj, carry):
            for p in range(2):
                j = jj * 2 + p
                pltpu.make_async_copy(table_hbm.at[idx_v.at[j]],
                                      bufs[p], sems[p]).wait()
                nxt = j + 1
                if True:
                    @pl.when(nxt < ENC)
                    def _():
                        pltpu.async_copy(table_hbm.at[idx_v.at[nxt]],
                                         bufs[(p + 1) % 2], sems[(p + 1) % 2])
                pltpu.sync_copy(bufs[p],
                                out_hbm.at[pl.ds(wid * PER + j * EC, EC)])
            return carry

        lax.fori_loop(0, ENC // 2, body, 0)

    return gather


def _msg1_body(xj_ref, ea_ref, w_ref, b_ref, out_ref):
    ea = ea_ref[...]
    wf = jnp.maximum(
        jnp.dot(ea, w_ref[...], preferred_element_type=jnp.float32)
        + b_ref[...], 0.0)
    xj = xj_ref[...]
    cols = []
    for o in range(H):
        p = wf[:, o * IN:(o + 1) * IN] * xj
        cols.append(jnp.sum(p, axis=1, keepdims=True))
    out_ref[...] = jnp.concatenate(cols, axis=1)


def _msg1(xj, ea, wr, br):
    BE = 512
    return pl.pallas_call(
        _msg1_body,
        grid=(EP // BE,),
        in_specs=[
            pl.BlockSpec((BE, IN), lambda i: (i, 0)),
            pl.BlockSpec((BE, 2), lambda i: (i, 0)),
            pl.BlockSpec((2, IN * H), lambda i: (0, 0)),
            pl.BlockSpec((1, IN * H), lambda i: (0, 0)),
        ],
        out_specs=pl.BlockSpec((BE, H), lambda i: (i, 0)),
        out_shape=jax.ShapeDtypeStruct((EP, H), jnp.float32),
    )(xj, ea, wr, br)


def _make_msg_small(oc, ow):
    def body(xj_ref, ea_ref, w_ref, b_ref, rep_ref, sel_ref, out_ref):
        wf = jnp.maximum(
            jnp.dot(ea_ref[...], w_ref[...],
                    preferred_element_type=jnp.float32) + b_ref[...], 0.0)
        xr = jnp.dot(xj_ref[:, :H], rep_ref[...],
                     preferred_element_type=jnp.float32)
        out_ref[...] = jnp.dot(wf * xr, sel_ref[...],
                               preferred_element_type=jnp.float32)

    BE = 2048
    K = H * oc

    def run(xj, ea, w, b, rep, sel):
        return pl.pallas_call(
            body,
            grid=(EP // BE,),
            in_specs=[
                pl.BlockSpec((BE, IN), lambda i: (i, 0)),
                pl.BlockSpec((BE, 2), lambda i: (i, 0)),
                pl.BlockSpec((2, K), lambda i: (0, 0)),
                pl.BlockSpec((1, K), lambda i: (0, 0)),
                pl.BlockSpec((H, K), lambda i: (0, 0)),
                pl.BlockSpec((K, ow), lambda i: (0, 0)),
            ],
            out_specs=pl.BlockSpec((BE, H), lambda i: (i, 0)),
            out_shape=jax.ShapeDtypeStruct((EP, H), jnp.float32),
        )(xj, ea, w, b, rep, sel)

    return run


_msg2 = _make_msg_small(H, H)
_msg3 = _make_msg_small(OUT, H)


def _agg_body(msg_ref, dstr_ref, dstc_ref, out_ref):
    i = pl.program_id(0)

    @pl.when(i == 0)
    def _():
        out_ref[...] = jnp.zeros((NACC, AW), jnp.float32)

    msg = msg_ref[...]
    m24 = jnp.concatenate(
        [msg, jnp.ones((BEA, 1), jnp.float32),
         jnp.zeros((BEA, AW - H - 1), jnp.float32)], axis=1)
    dr = dstr_ref[0]
    dc = dstc_ref[0]
    lo = jax.lax.rem(dr, LO)
    ohT = (jax.lax.broadcasted_iota(jnp.int32, (LO, BEA), 0) ==
           lo).astype(jnp.float32)
    hic = dc // LO
    x = jnp.concatenate(
        [m24 * (hic == hi).astype(jnp.float32) for hi in range(NHI)], axis=1)
    res = jnp.dot(ohT, x, preferred_element_type=jnp.float32)
    for hi in range(NHI):
        out_ref[pl.ds(hi * LO, LO), :] += res[:, hi * AW:(hi + 1) * AW]


def _agg(msg, dstr, dstc):
    return pl.pallas_call(
        _agg_body,
        grid=(EP // BEA,),
        in_specs=[
            pl.BlockSpec((BEA, H), lambda i: (i, 0)),
            pl.BlockSpec((1, 1, BEA), lambda i: (i, 0, 0)),
            pl.BlockSpec((1, BEA, 1), lambda i: (i, 0, 0)),
        ],
        out_specs=pl.BlockSpec((NACC, AW), lambda i: (0, 0)),
        out_shape=jax.ShapeDtypeStruct((NACC, AW), jnp.float32),
    )(msg, dstr, dstc)


def _make_finalize(ric, oc, do_relu, ow):
    R = 1024

    def body(acc_ref, hp_ref, root_ref, bias_ref, out_ref):
        acc = acc_ref[...]
        cnt = acc[:, H:H + 1]
        agg = acc[:, :oc] / jnp.maximum(cnt, 1.0)
        h = agg + jnp.dot(hp_ref[:, :ric], root_ref[...],
                          preferred_element_type=jnp.float32) + bias_ref[...]
        h = jnp.maximum(h, 0.0) if do_relu else h
        if ow > oc:
            h = jnp.concatenate(
                [h, jnp.zeros((h.shape[0], ow - oc), jnp.float32)], axis=1)
        out_ref[...] = h

    def run(acc, hp, root, bias):
        return pl.pallas_call(
            body,
            grid=(NACC // R,),
            in_specs=[
                pl.BlockSpec((R, AW), lambda i: (i, 0)),
                pl.BlockSpec((R, IN), lambda i: (i, 0)),
                pl.BlockSpec((ric, oc), lambda i: (0, 0)),
                pl.BlockSpec((1, oc), lambda i: (0, 0)),
            ],
            out_specs=pl.BlockSpec((R, ow), lambda i: (i, 0)),
            out_shape=jax.ShapeDtypeStruct((NACC, ow), jnp.float32),
        )(acc, hp, root, bias)

    return run


_fin1 = _make_finalize(IN, H, True, IN)
_fin2 = _make_finalize(H, H, True, IN)
_fin3 = _make_finalize(H, OUT, False, OUT)

_REP2 = np.kron(np.eye(H), np.ones((1, H))).astype(np.float32)
_SEL2 = np.kron(np.ones((H, 1)), np.eye(H)).astype(np.float32)
_REP3 = np.kron(np.eye(H), np.ones((1, OUT))).astype(np.float32)
_SEL3 = np.zeros((H * OUT, H), np.float32)
_SEL3[:, :OUT] = np.kron(np.ones((H, 1)), np.eye(OUT))


def kernel(x, edge_index, edge_attr, W1, b1, W2, b2, W3, b3,
           root1, bias1, root2, bias2, root3, bias3):
    src = jnp.pad(edge_index[0], (0, EP - E)).reshape(NW, ENC, EC)
    dst_flat = jnp.pad(edge_index[1], (0, EP - E), constant_values=N)
    dstr = dst_flat.reshape(EP // BEA, 1, BEA)
    dstc = dst_flat.reshape(EP // BEA, BEA, 1)
    ea = jnp.pad(edge_attr, ((0, EP - E), (0, 0)))
    x_pad = jnp.pad(x, ((0, NACC - N), (0, 0)))

    wr1 = W1.reshape(2, IN, H).transpose(0, 2, 1).reshape(2, IN * H)
    br1 = b1.reshape(IN, H).T.reshape(1, IN * H)

    _gather128 = _make_gather(IN)

    xj = _gather128(x, src)
    m1 = _msg1(xj, ea, wr1, br1)
    acc1 = _agg(m1, dstr, dstc)
    h1 = _fin1(acc1, x_pad, root1, bias1.reshape(1, H))

    xj2 = _gather128(h1, src)
    m2 = _msg2(xj2, ea, W2, b2.reshape(1, H * H), _REP2, _SEL2)
    acc2 = _agg(m2, dstr, dstc)
    h2 = _fin2(acc2, h1, root2, bias2.reshape(1, H))

    xj3 = _gather128(h2, src)
    m3 = _msg3(xj3, ea, W3, b3.reshape(1, H * OUT), _REP3, _SEL3)
    acc3 = _agg(m3, dstr, dstc)
    out = _fin3(acc3, h2, root3, bias3.reshape(1, OUT))
    return out[:N]
